# baseline (speedup 1.0000x reference)
import functools

import jax
import jax.numpy as jnp
from jax import lax
from jax.experimental import pallas as pl
from jax.experimental.pallas import tpu as pltpu
from jax.experimental.pallas import tpu_sc as plsc

_NC, _NS, _L = 2, 16, 16


def _sc_kernel(vectors, attributes, lo=0, cnt=16):
    n, f = vectors.shape
    a_cnt = attributes.shape[0]
    vt = vectors.T
    nw = _NC * _NS
    wpp = nw // cnt
    rows_per_w = n // wpp
    blk = 16
    nblk = rows_per_w // blk
    nj = n // _L
    mesh = plsc.VectorSubcoreMesh(core_axis_name="c", subcore_axis_name="s")

    @functools.partial(
        pl.kernel,
        mesh=mesh,
        out_type=jax.ShapeDtypeStruct((cnt, n, n), jnp.float32),
        scratch_types=[
            pltpu.VMEM((a_cnt,), jnp.int32),
            pltpu.VMEM((a_cnt, n), jnp.float32),
            pltpu.VMEM((2, blk, n), jnp.float32),
            pltpu.SemaphoreType.DMA,
            pltpu.SemaphoreType.DMA,
            pltpu.SemaphoreType.DMA,
        ],
    )
    def k(vt_hbm, attrs_hbm, out_hbm, attrs_v, cols_v, buf_v, gsem, osem0, osem1):
        wid = lax.axis_index("s") * _NC + lax.axis_index("c")
        ai = lo + wid // wpp
        row0 = (wid % wpp) * rows_per_w
        osems = (osem0, osem1)
        pltpu.sync_copy(attrs_hbm, attrs_v)
        pltpu.async_copy(vt_hbm.at[attrs_v], cols_v, gsem).wait()

        def compute_block(i0, s):
            vchunk = cols_v[ai, pl.ds(i0, blk)]
            splats = [jnp.full((_L,), vchunk[l], jnp.float32) for l in range(blk)]

            def j_body(jc, _):
                v = cols_v[ai, pl.ds(jc * _L, _L)]
                for l in range(blk):
                    buf_v[s, l, pl.ds(jc * _L, _L)] = jnp.abs(v - splats[l])
                return 0

            lax.fori_loop(0, nj, j_body, 0)

        def outer(bb, _):
            for s in range(2):
                i0 = row0 + (bb * 2 + s) * blk

                @pl.when(bb > 0)
                def _():
                    pltpu.make_async_copy(
                        buf_v.at[s], out_hbm.at[ai - lo, pl.ds(i0, blk)], osems[s]
                    ).wait()

                compute_block(i0, s)
                pltpu.async_copy(
                    buf_v.at[s], out_hbm.at[ai - lo, pl.ds(i0, blk)], osems[s]
                )
            return 0

        lax.fori_loop(0, nblk // 2, outer, 0)
        for s in range(2):
            pltpu.make_async_copy(
                buf_v.at[s], out_hbm.at[ai - lo, pl.ds(row0, blk)], osems[s]
            ).wait()

    return k(vt, attributes.astype(jnp.int32))


def _body(attrs_ref, vt_ref, out_ref, *, block_i: int):
    ai = pl.program_id(0)
    i = pl.program_id(1)
    attr = attrs_ref[ai]
    col = vt_ref[pl.ds(attr, 1), :]
    rows = vt_ref[pl.ds(attr, 1), pl.ds(i * block_i, block_i)]
    out_ref[0, :, :] = jnp.abs(rows[0][:, None] - col)


def _body_xpose(attrs_ref, vec_ref, out_ref, vt_s, *, block_i: int):
    ai = pl.program_id(0)
    i = pl.program_id(1)

    @pl.when((ai == 0) & (i == 0))
    def _():
        n, f = vec_ref.shape
        for c in range(n // 128):
            vt_s[:, c * 128:(c + 1) * 128] = vec_ref[c * 128:(c + 1) * 128, :].T

    attr = attrs_ref[ai]
    col = vt_s[pl.ds(attr, 1), :]
    rows = vt_s[pl.ds(attr, 1), pl.ds(i * block_i, block_i)]
    out_ref[0, :, :] = jnp.abs(rows[0][:, None] - col)


def _tc_kernel_xp(vectors, attributes):
    n, f = vectors.shape
    a = attributes.shape[0]
    block_i = 512
    grid = (a, n // block_i)

    body = functools.partial(_body_xpose, block_i=block_i)
    out = pl.pallas_call(
        body,
        grid=grid,
        in_specs=[
            pl.BlockSpec(memory_space=pltpu.SMEM),
            pl.BlockSpec((n, f), lambda ai, i: (0, 0)),
        ],
        out_specs=pl.BlockSpec((1, block_i, n), lambda ai, i: (ai, i, 0)),
        out_shape=jax.ShapeDtypeStruct((a, n, n), jnp.float32),
        scratch_shapes=[pltpu.VMEM((f, n), jnp.float32)],
        compiler_params=pltpu.CompilerParams(
            dimension_semantics=("arbitrary", "arbitrary"),
        ),
    )(attributes.astype(jnp.int32), vectors)
    return out


def _tc_kernel(vectors, attributes, cnt=None):
    n, f = vectors.shape
    a = attributes.shape[0] if cnt is None else cnt
    vt = vectors.T
    block_i = 512
    grid = (a, n // block_i)

    body = functools.partial(_body, block_i=block_i)
    out = pl.pallas_call(
        body,
        grid=grid,
        in_specs=[
            pl.BlockSpec(memory_space=pltpu.SMEM),
            pl.BlockSpec((f, n), lambda ai, i: (0, 0)),
        ],
        out_specs=pl.BlockSpec((1, block_i, n), lambda ai, i: (ai, i, 0)),
        out_shape=jax.ShapeDtypeStruct((a, n, n), jnp.float32),
        compiler_params=pltpu.CompilerParams(
            dimension_semantics=("parallel", "parallel"),
        ),
    )(attributes.astype(jnp.int32), vt)
    return out


def _hybrid_kernel(vectors, attributes, sc_cnt=2):
    a = attributes.shape[0]
    tc_out = _tc_kernel(vectors, attributes, cnt=a - sc_cnt)
    sc_out = _sc_kernel(vectors, attributes, lo=a - sc_cnt, cnt=sc_cnt)
    return jnp.concatenate([tc_out, sc_out], axis=0)


kernel = _sc_kernel


# speedup vs baseline: 1.6946x; 1.6946x over previous
import functools

import jax
import jax.numpy as jnp
from jax.experimental import pallas as pl
from jax.experimental.pallas import tpu as pltpu


def _body(attrs_ref, vec_ref, out_ref, vt_s, *, block_i: int):
    ai = pl.program_id(0)
    i = pl.program_id(1)

    @pl.when((ai == 0) & (i == 0))
    def _():
        n, f = vec_ref.shape
        for c in range(n // 128):
            vt_s[:, c * 128:(c + 1) * 128] = vec_ref[c * 128:(c + 1) * 128, :].T

    attr = attrs_ref[ai]
    col = vt_s[pl.ds(attr, 1), :]
    rows = vt_s[pl.ds(attr, 1), pl.ds(i * block_i, block_i)]
    out_ref[0, :, :] = jnp.abs(rows[0][:, None] - col)


def kernel(vectors, attributes):
    n, f = vectors.shape
    a = attributes.shape[0]
    block_i = 512
    grid = (a, n // block_i)

    body = functools.partial(_body, block_i=block_i)
    out = pl.pallas_call(
        body,
        grid=grid,
        in_specs=[
            pl.BlockSpec(memory_space=pltpu.SMEM),
            pl.BlockSpec((n, f), lambda ai, i: (0, 0)),
        ],
        out_specs=pl.BlockSpec((1, block_i, n), lambda ai, i: (ai, i, 0)),
        out_shape=jax.ShapeDtypeStruct((a, n, n), jnp.float32),
        scratch_shapes=[pltpu.VMEM((f, n), jnp.float32)],
        compiler_params=pltpu.CompilerParams(
            dimension_semantics=("arbitrary", "arbitrary"),
        ),
    )(attributes.astype(jnp.int32), vectors)
    return out
